# baseline (speedup 1.0000x reference)
import functools

import jax
import jax.numpy as jnp
from jax import lax
from jax.experimental import pallas as pl
from jax.experimental.pallas import tpu as pltpu
from jax.experimental.pallas import tpu_sc as plsc

_N = 8192
_E = 8192
_D = 32

_RB = 512
_CB = 2048

_ERB = 1024
_ECB = 2048


def _dist_argmin_kernel(z_ref, cb_ref, zsq_ref, esq_ref,
                        dist_ref, idx_ref, min_scr, arg_scr):
    j = pl.program_id(1)
    ncols = pl.num_programs(1)
    z = z_ref[...]
    cb = cb_ref[...]
    mm = jax.lax.dot_general(z, cb, (((1,), (1,)), ((), ())),
                             preferred_element_type=jnp.float32)
    d = (zsq_ref[...] + esq_ref[...]) - 2.0 * mm
    dist_ref[...] = d

    lmin = jnp.min(d, axis=1, keepdims=True)
    col = jax.lax.broadcasted_iota(jnp.int32, (_RB, _CB), 1) + j * _CB
    larg = jnp.min(jnp.where(d == lmin, col, jnp.int32(2**30)),
                   axis=1, keepdims=True)

    @pl.when(j == 0)
    def _init():
        min_scr[...] = lmin
        arg_scr[...] = larg

    @pl.when(j > 0)
    def _update():
        better = lmin < min_scr[...]
        arg_scr[...] = jnp.where(better, larg, arg_scr[...])
        min_scr[...] = jnp.where(better, lmin, min_scr[...])

    @pl.when(j == ncols - 1)
    def _emit():
        idx_ref[...] = arg_scr[...]


def _encodings_kernel(idx_ref, enc_ref):
    j = pl.program_id(1)
    col = jax.lax.broadcasted_iota(jnp.int32, (_ERB, _ECB), 1) + j * _ECB
    enc_ref[...] = (idx_ref[...] == col).astype(jnp.float32)


_NC = 2
_NS = 16
_NW = _NC * _NS
_RPW = _N // _NW
_ZW = 65536
_NCOPY = (_RPW * _E) // _ZW
_BURST = 16


def _enc_sc_body(idx_hbm, out_hbm, zbuf, idxv, ones_v, zsem, ssem):
    wid = lax.axis_index("s") * _NC + lax.axis_index("c")
    base_row = wid * _RPW
    base_elem = base_row * _E

    zero16 = jnp.zeros((16,), jnp.float32)

    def _zinit(i, carry):
        for u in range(8):
            zbuf[pl.ds((i * 8 + u) * 16, 16)] = zero16
        return carry

    lax.fori_loop(0, _ZW // 128, _zinit, 0)
    ones_v[...] = jnp.full((16,), 1.0, jnp.float32)
    pltpu.sync_copy(idx_hbm.at[pl.ds(base_row, _RPW)], idxv)

    def _zfill(g, carry):
        for u in range(_BURST):
            pltpu.async_copy(
                zbuf,
                out_hbm.at[pl.ds(base_elem + (g * _BURST + u) * _ZW, _ZW)],
                zsem)
        for u in range(_BURST):
            pltpu.make_async_copy(
                zbuf,
                out_hbm.at[pl.ds(base_elem + (g * _BURST + u) * _ZW, _ZW)],
                zsem).wait()
        return carry

    lax.fori_loop(0, _NCOPY // _BURST, _zfill, 0)

    iota16 = lax.iota(jnp.int32, 16)
    for t in range(_RPW // 16):
        idx16 = idxv[pl.ds(t * 16, 16)]
        flat = (iota16 + (base_row + t * 16)) * _E + idx16
        pltpu.async_copy(ones_v, out_hbm.at[flat], ssem).wait()


_enc_sc = functools.partial(
    pl.kernel,
    out_type=jax.ShapeDtypeStruct((_N * _E,), jnp.float32),
    mesh=plsc.VectorSubcoreMesh(
        core_axis_name="c", subcore_axis_name="s",
        num_cores=_NC, num_subcores=_NS),
    scratch_types=[
        pltpu.VMEM((_ZW,), jnp.float32),
        pltpu.VMEM((_RPW,), jnp.int32),
        pltpu.VMEM((16,), jnp.float32),
        pltpu.SemaphoreType.DMA,
        pltpu.SemaphoreType.DMA,
    ],
)(_enc_sc_body)


def kernel(z_e, codebook):
    z_e_nhwc = jnp.transpose(z_e, (0, 2, 3, 1))
    z_flat = z_e_nhwc.reshape(-1, _D)
    z_sq = jnp.sum(z_flat ** 2, axis=1, keepdims=True)
    e_sq = jnp.sum(codebook ** 2, axis=1).reshape(1, _E)

    dist, idx2d = pl.pallas_call(
        _dist_argmin_kernel,
        grid=(_N // _RB, _E // _CB),
        in_specs=[
            pl.BlockSpec((_RB, _D), lambda i, j: (i, 0)),
            pl.BlockSpec((_CB, _D), lambda i, j: (j, 0)),
            pl.BlockSpec((_RB, 1), lambda i, j: (i, 0)),
            pl.BlockSpec((1, _CB), lambda i, j: (0, j)),
        ],
        out_specs=[
            pl.BlockSpec((_RB, _CB), lambda i, j: (i, j)),
            pl.BlockSpec((_RB, 1), lambda i, j: (i, 0)),
        ],
        out_shape=[
            jax.ShapeDtypeStruct((_N, _E), jnp.float32),
            jax.ShapeDtypeStruct((_N, 1), jnp.int32),
        ],
        scratch_shapes=[
            pltpu.VMEM((_RB, 1), jnp.float32),
            pltpu.VMEM((_RB, 1), jnp.int32),
        ],
    )(z_flat, codebook, z_sq, e_sq)

    indices = idx2d.reshape(_N)
    encodings = _enc_sc(indices).reshape(_N, _E)
    return (z_e_nhwc, z_flat, dist, indices, encodings)


# speedup vs baseline: 2.5283x; 2.5283x over previous
import jax
import jax.numpy as jnp
from jax.experimental import pallas as pl
from jax.experimental.pallas import tpu as pltpu

_N = 8192
_E = 8192
_D = 32

_RB = 1024
_CB = 2048

_ERB = 2048
_ECB = 2048


def _dist_argmin_kernel(z_ref, cb_ref, zsq_ref, esq_ref,
                        dist_ref, idx_ref, min_scr, arg_scr):
    j = pl.program_id(1)
    ncols = pl.num_programs(1)
    z = z_ref[...]
    cb = cb_ref[...]
    mm = jax.lax.dot_general(z, cb, (((1,), (1,)), ((), ())),
                             preferred_element_type=jnp.float32)
    d = (zsq_ref[...] + esq_ref[...]) - 2.0 * mm
    dist_ref[...] = d

    lmin = jnp.min(d, axis=1, keepdims=True)
    col = jax.lax.broadcasted_iota(jnp.int32, (_RB, _CB), 1) + j * _CB
    larg = jnp.min(jnp.where(d == lmin, col, jnp.int32(2**30)),
                   axis=1, keepdims=True)

    @pl.when(j == 0)
    def _init():
        min_scr[...] = lmin
        arg_scr[...] = larg

    @pl.when(j > 0)
    def _update():
        better = lmin < min_scr[...]
        arg_scr[...] = jnp.where(better, larg, arg_scr[...])
        min_scr[...] = jnp.where(better, lmin, min_scr[...])

    @pl.when(j == ncols - 1)
    def _emit():
        idx_ref[...] = arg_scr[...]


def _encodings_kernel(idx_ref, enc_ref):
    j = pl.program_id(1)
    col = jax.lax.broadcasted_iota(jnp.int32, (_ERB, _ECB), 1) + j * _ECB
    enc_ref[...] = (idx_ref[...] == col).astype(jnp.float32)


def kernel(z_e, codebook):
    z_e_nhwc = jnp.transpose(z_e, (0, 2, 3, 1))
    z_flat = z_e_nhwc.reshape(-1, _D)
    z_sq = jnp.sum(z_flat ** 2, axis=1, keepdims=True)
    e_sq = jnp.sum(codebook ** 2, axis=1).reshape(1, _E)

    dist, idx2d = pl.pallas_call(
        _dist_argmin_kernel,
        grid=(_N // _RB, _E // _CB),
        in_specs=[
            pl.BlockSpec((_RB, _D), lambda i, j: (i, 0)),
            pl.BlockSpec((_CB, _D), lambda i, j: (j, 0)),
            pl.BlockSpec((_RB, 1), lambda i, j: (i, 0)),
            pl.BlockSpec((1, _CB), lambda i, j: (0, j)),
        ],
        out_specs=[
            pl.BlockSpec((_RB, _CB), lambda i, j: (i, j)),
            pl.BlockSpec((_RB, 1), lambda i, j: (i, 0)),
        ],
        out_shape=[
            jax.ShapeDtypeStruct((_N, _E), jnp.float32),
            jax.ShapeDtypeStruct((_N, 1), jnp.int32),
        ],
        scratch_shapes=[
            pltpu.VMEM((_RB, 1), jnp.float32),
            pltpu.VMEM((_RB, 1), jnp.int32),
        ],
    )(z_flat, codebook, z_sq, e_sq)

    encodings = pl.pallas_call(
        _encodings_kernel,
        grid=(_N // _ERB, _E // _ECB),
        in_specs=[pl.BlockSpec((_ERB, 1), lambda i, j: (i, 0))],
        out_specs=pl.BlockSpec((_ERB, _ECB), lambda i, j: (i, j)),
        out_shape=jax.ShapeDtypeStruct((_N, _E), jnp.float32),
    )(idx2d)

    indices = idx2d.reshape(_N)
    return (z_e_nhwc, z_flat, dist, indices, encodings)


# speedup vs baseline: 2.5953x; 1.0265x over previous
import jax
import jax.numpy as jnp
from jax.experimental import pallas as pl
from jax.experimental.pallas import tpu as pltpu

_N = 8192
_E = 8192
_D = 32

_RB = 2048
_CB = 2048

_ERB = 2048
_ECB = 2048


def _dist_argmin_kernel(z_ref, cb_ref, zsq_ref, esq_ref,
                        dist_ref, idx_ref, min_scr, arg_scr):
    j = pl.program_id(1)
    ncols = pl.num_programs(1)
    z = z_ref[...]
    cb = cb_ref[...]
    mm = jax.lax.dot_general(z, cb, (((1,), (1,)), ((), ())),
                             preferred_element_type=jnp.float32)
    d = (zsq_ref[...] + esq_ref[...]) - 2.0 * mm
    dist_ref[...] = d

    lmin = jnp.min(d, axis=1, keepdims=True)
    col = jax.lax.broadcasted_iota(jnp.int32, (_RB, _CB), 1) + j * _CB
    larg = jnp.min(jnp.where(d == lmin, col, jnp.int32(2**30)),
                   axis=1, keepdims=True)

    @pl.when(j == 0)
    def _init():
        min_scr[...] = lmin
        arg_scr[...] = larg

    @pl.when(j > 0)
    def _update():
        better = lmin < min_scr[...]
        arg_scr[...] = jnp.where(better, larg, arg_scr[...])
        min_scr[...] = jnp.where(better, lmin, min_scr[...])

    @pl.when(j == ncols - 1)
    def _emit():
        idx_ref[...] = arg_scr[...]


def _encodings_kernel(idx_ref, enc_ref):
    j = pl.program_id(1)
    col = jax.lax.broadcasted_iota(jnp.int32, (_ERB, _ECB), 1) + j * _ECB
    enc_ref[...] = (idx_ref[...] == col).astype(jnp.float32)


def kernel(z_e, codebook):
    z_e_nhwc = jnp.transpose(z_e, (0, 2, 3, 1))
    z_flat = z_e_nhwc.reshape(-1, _D)
    z_sq = jnp.sum(z_flat ** 2, axis=1, keepdims=True)
    e_sq = jnp.sum(codebook ** 2, axis=1).reshape(1, _E)

    dist, idx2d = pl.pallas_call(
        _dist_argmin_kernel,
        grid=(_N // _RB, _E // _CB),
        in_specs=[
            pl.BlockSpec((_RB, _D), lambda i, j: (i, 0)),
            pl.BlockSpec((_CB, _D), lambda i, j: (j, 0)),
            pl.BlockSpec((_RB, 1), lambda i, j: (i, 0)),
            pl.BlockSpec((1, _CB), lambda i, j: (0, j)),
        ],
        out_specs=[
            pl.BlockSpec((_RB, _CB), lambda i, j: (i, j)),
            pl.BlockSpec((_RB, 1), lambda i, j: (i, 0)),
        ],
        out_shape=[
            jax.ShapeDtypeStruct((_N, _E), jnp.float32),
            jax.ShapeDtypeStruct((_N, 1), jnp.int32),
        ],
        scratch_shapes=[
            pltpu.VMEM((_RB, 1), jnp.float32),
            pltpu.VMEM((_RB, 1), jnp.int32),
        ],
    )(z_flat, codebook, z_sq, e_sq)

    encodings = pl.pallas_call(
        _encodings_kernel,
        grid=(_N // _ERB, _E // _ECB),
        in_specs=[pl.BlockSpec((_ERB, 1), lambda i, j: (i, 0))],
        out_specs=pl.BlockSpec((_ERB, _ECB), lambda i, j: (i, j)),
        out_shape=jax.ShapeDtypeStruct((_N, _E), jnp.float32),
    )(idx2d)

    indices = idx2d.reshape(_N)
    return (z_e_nhwc, z_flat, dist, indices, encodings)


# speedup vs baseline: 2.8122x; 1.0836x over previous
import jax
import jax.numpy as jnp
from jax.experimental import pallas as pl

_N = 8192
_E = 8192
_D = 32

_RB = 512
_ERB = 512


def _dist_argmin_kernel(z_ref, cb_ref, zsq_ref, esq_ref, dist_ref, idx_ref):
    z = z_ref[...]
    cb = cb_ref[...]
    mm = jax.lax.dot_general(z, cb, (((1,), (1,)), ((), ())),
                             preferred_element_type=jnp.float32)
    d = (zsq_ref[...] + esq_ref[...]) - 2.0 * mm
    dist_ref[...] = d

    lmin = jnp.min(d, axis=1, keepdims=True)
    col = jax.lax.broadcasted_iota(jnp.int32, (_RB, _E), 1)
    idx_ref[...] = jnp.min(jnp.where(d == lmin, col, jnp.int32(2**30)),
                           axis=1, keepdims=True)


def _encodings_kernel(idx_ref, enc_ref):
    col = jax.lax.broadcasted_iota(jnp.int32, (_ERB, _E), 1)
    enc_ref[...] = (idx_ref[...] == col).astype(jnp.float32)


def kernel(z_e, codebook):
    z_e_nhwc = jnp.transpose(z_e, (0, 2, 3, 1))
    z_flat = z_e_nhwc.reshape(-1, _D)
    z_sq = jnp.sum(z_flat ** 2, axis=1, keepdims=True)
    e_sq = jnp.sum(codebook ** 2, axis=1).reshape(1, _E)

    dist, idx2d = pl.pallas_call(
        _dist_argmin_kernel,
        grid=(_N // _RB,),
        in_specs=[
            pl.BlockSpec((_RB, _D), lambda i: (i, 0)),
            pl.BlockSpec((_E, _D), lambda i: (0, 0)),
            pl.BlockSpec((_RB, 1), lambda i: (i, 0)),
            pl.BlockSpec((1, _E), lambda i: (0, 0)),
        ],
        out_specs=[
            pl.BlockSpec((_RB, _E), lambda i: (i, 0)),
            pl.BlockSpec((_RB, 1), lambda i: (i, 0)),
        ],
        out_shape=[
            jax.ShapeDtypeStruct((_N, _E), jnp.float32),
            jax.ShapeDtypeStruct((_N, 1), jnp.int32),
        ],
    )(z_flat, codebook, z_sq, e_sq)

    encodings = pl.pallas_call(
        _encodings_kernel,
        grid=(_N // _ERB,),
        in_specs=[pl.BlockSpec((_ERB, 1), lambda i: (i, 0))],
        out_specs=pl.BlockSpec((_ERB, _E), lambda i: (i, 0)),
        out_shape=jax.ShapeDtypeStruct((_N, _E), jnp.float32),
    )(idx2d)

    indices = idx2d.reshape(_N)
    return (z_e_nhwc, z_flat, dist, indices, encodings)


# speedup vs baseline: 2.8656x; 1.0190x over previous
import jax
import jax.numpy as jnp
from jax.experimental import pallas as pl
from jax.experimental.pallas import tpu as pltpu

_N = 8192
_E = 8192
_D = 32

_RB = 256
_NS = _N // _RB


def _vq_kernel(z_ref, cb_ref, zsq_ref, esq_ref,
               dist_ref, idx_ref, enc_ref, idx_scr):
    i = pl.program_id(0)

    @pl.when(i > 0)
    def _emit_enc():
        col = jax.lax.broadcasted_iota(jnp.int32, (_RB, _E), 1)
        enc_ref[...] = (idx_scr[...] == col).astype(jnp.float32)

    @pl.when(i < _NS)
    def _dist_step():
        z = z_ref[...]
        cb = cb_ref[...]
        mm = jax.lax.dot_general(z, cb, (((1,), (1,)), ((), ())),
                                 preferred_element_type=jnp.float32)
        d = (zsq_ref[...] + esq_ref[...]) - 2.0 * mm
        dist_ref[...] = d

        lmin = jnp.min(d, axis=1, keepdims=True)
        col = jax.lax.broadcasted_iota(jnp.int32, (_RB, _E), 1)
        larg = jnp.min(jnp.where(d == lmin, col, jnp.int32(2**30)),
                       axis=1, keepdims=True)
        idx_ref[...] = larg
        idx_scr[...] = larg


def kernel(z_e, codebook):
    z_e_nhwc = jnp.transpose(z_e, (0, 2, 3, 1))
    z_flat = z_e_nhwc.reshape(-1, _D)
    z_sq = jnp.sum(z_flat ** 2, axis=1, keepdims=True)
    e_sq = jnp.sum(codebook ** 2, axis=1).reshape(1, _E)

    last = _NS - 1
    dist, idx2d, encodings = pl.pallas_call(
        _vq_kernel,
        grid=(_NS + 1,),
        in_specs=[
            pl.BlockSpec((_RB, _D), lambda i: (jnp.minimum(i, last), 0)),
            pl.BlockSpec((_E, _D), lambda i: (0, 0)),
            pl.BlockSpec((_RB, 1), lambda i: (jnp.minimum(i, last), 0)),
            pl.BlockSpec((1, _E), lambda i: (0, 0)),
        ],
        out_specs=[
            pl.BlockSpec((_RB, _E), lambda i: (jnp.minimum(i, last), 0)),
            pl.BlockSpec((_RB, 1), lambda i: (jnp.minimum(i, last), 0)),
            pl.BlockSpec((_RB, _E), lambda i: (jnp.maximum(i - 1, 0), 0)),
        ],
        out_shape=[
            jax.ShapeDtypeStruct((_N, _E), jnp.float32),
            jax.ShapeDtypeStruct((_N, 1), jnp.int32),
            jax.ShapeDtypeStruct((_N, _E), jnp.float32),
        ],
        scratch_shapes=[pltpu.VMEM((_RB, 1), jnp.int32)],
    )(z_flat, codebook, z_sq, e_sq)

    indices = idx2d.reshape(_N)
    return (z_e_nhwc, z_flat, dist, indices, encodings)
